# baseline (device time: 172166 ns/iter reference)
import jax
import jax.numpy as jnp
from jax import lax
from jax.experimental import pallas as pl
from jax.experimental.pallas import tpu as pltpu

NZ = 4


def kernel(O, Wo):
    B, S, Hs, D = O.shape
    K = Hs * D
    N = Wo.shape[1]
    S_out = S // NZ

    p = jnp.dot(
        O.reshape(B * S, K).astype(jnp.bfloat16),
        Wo.astype(jnp.bfloat16),
        preferred_element_type=jnp.bfloat16,
    ).reshape(B, S, N)

    def body(p_ref, out_ref, acc_ref, recv_ref, send_sems, recv_sems):
        x = lax.axis_index("x")
        y = lax.axis_index("y")
        z = lax.axis_index("z")
        right = (z + 1) % NZ
        left = (z + NZ - 1) % NZ

        barrier = pltpu.get_barrier_semaphore()
        for nbr in (left, right):
            pl.semaphore_signal(
                barrier, inc=1,
                device_id=(x, y, nbr),
                device_id_type=pl.DeviceIdType.MESH,
            )
        pl.semaphore_wait(barrier, 2)

        c0 = (z + NZ - 1) % NZ
        acc_ref[...] = p_ref[:, pl.ds(c0 * S_out, S_out), :]

        for s in range(NZ - 1):
            rdma = pltpu.make_async_remote_copy(
                src_ref=acc_ref,
                dst_ref=recv_ref.at[s],
                send_sem=send_sems.at[s],
                recv_sem=recv_sems.at[s],
                device_id=(x, y, right),
                device_id_type=pl.DeviceIdType.MESH,
            )
            rdma.start()
            rdma.wait()

            ridx = (z + NZ - 2 - s) % NZ
            chunk = p_ref[:, pl.ds(ridx * S_out, S_out), :]
            if s < NZ - 2:
                acc_ref[...] = recv_ref[s] + chunk
            else:
                out_ref[...] = (
                    recv_ref[s].astype(jnp.float32)
                    + chunk.astype(jnp.float32)
                )

    return pl.pallas_call(
        body,
        out_shape=jax.ShapeDtypeStruct((B, S_out, N), jnp.float32),
        in_specs=[pl.BlockSpec(memory_space=pltpu.VMEM)],
        out_specs=pl.BlockSpec(memory_space=pltpu.VMEM),
        scratch_shapes=[
            pltpu.VMEM((B, S_out, N), jnp.bfloat16),
            pltpu.VMEM((NZ - 1, B, S_out, N), jnp.bfloat16),
            pltpu.SemaphoreType.DMA((NZ - 1,)),
            pltpu.SemaphoreType.DMA((NZ - 1,)),
        ],
        compiler_params=pltpu.CompilerParams(collective_id=0),
    )(p)


# device time: 166369 ns/iter; 1.0348x vs baseline; 1.0348x over previous
import jax
import jax.numpy as jnp
from jax import lax
from jax.experimental import pallas as pl
from jax.experimental.pallas import tpu as pltpu

NZ = 4


def kernel(O, Wo):
    B, S, Hs, D = O.shape
    K = Hs * D
    N = Wo.shape[1]
    S_out = S // NZ
    NSTEP = NZ - 1

    O_bf = O.reshape(B, S, K).astype(jnp.bfloat16)
    Wo_bf = Wo.astype(jnp.bfloat16)

    def body(o_ref, w_ref, out_ref, acc_ref, pc_ref, recv_ref,
             send_sems, recv_sems):
        x = lax.axis_index("x")
        y = lax.axis_index("y")
        z = lax.axis_index("z")
        right = (z + 1) % NZ
        left = (z + NZ - 1) % NZ

        barrier = pltpu.get_barrier_semaphore()
        for nbr in (left, right):
            pl.semaphore_signal(
                barrier, inc=1,
                device_id=(x, y, nbr),
                device_id_type=pl.DeviceIdType.MESH,
            )
        pl.semaphore_wait(barrier, 2)

        def pmm(c, b):
            return jnp.dot(
                o_ref[b, pl.ds(c * S_out, S_out), :], w_ref[...],
                preferred_element_type=jnp.float32,
            ).astype(jnp.bfloat16)

        def mk(s, b):
            return pltpu.make_async_remote_copy(
                src_ref=acc_ref.at[s, b],
                dst_ref=recv_ref.at[s, b],
                send_sem=send_sems.at[s, b],
                recv_sem=recv_sems.at[s, b],
                device_id=(x, y, right),
                device_id_type=pl.DeviceIdType.MESH,
            )

        rdmas = {}
        c0 = (z + NZ - 1) % NZ
        for b in range(B):
            acc_ref[0, b] = pmm(c0, b)
            rdmas[(0, b)] = mk(0, b)
            rdmas[(0, b)].start()

        for s in range(NSTEP):
            ridx = (z + NZ - 2 - s) % NZ
            for b in range(B):
                pc_ref[s, b] = pmm(ridx, b)

        for s in range(NSTEP):
            for b in range(B):
                rdmas[(s, b)].wait_recv()
                if s < NSTEP - 1:
                    acc_ref[s + 1, b] = recv_ref[s, b] + pc_ref[s, b]
                    rdmas[(s + 1, b)] = mk(s + 1, b)
                    rdmas[(s + 1, b)].start()
                else:
                    out_ref[b] = (
                        recv_ref[s, b].astype(jnp.float32)
                        + pc_ref[s, b].astype(jnp.float32)
                    )

        for s in range(NSTEP):
            for b in range(B):
                rdmas[(s, b)].wait_send()

    return pl.pallas_call(
        body,
        out_shape=jax.ShapeDtypeStruct((B, S_out, N), jnp.float32),
        in_specs=[
            pl.BlockSpec(memory_space=pltpu.VMEM),
            pl.BlockSpec(memory_space=pltpu.VMEM),
        ],
        out_specs=pl.BlockSpec(memory_space=pltpu.VMEM),
        scratch_shapes=[
            pltpu.VMEM((NSTEP, B, S_out, N), jnp.bfloat16),
            pltpu.VMEM((NSTEP, B, S_out, N), jnp.bfloat16),
            pltpu.VMEM((NSTEP, B, S_out, N), jnp.bfloat16),
            pltpu.SemaphoreType.DMA((NSTEP, B)),
            pltpu.SemaphoreType.DMA((NSTEP, B)),
        ],
        compiler_params=pltpu.CompilerParams(
            collective_id=0,
            vmem_limit_bytes=100 * 1024 * 1024,
        ),
    )(O_bf, Wo_bf)
